# Initial kernel scaffold; baseline (speedup 1.0000x reference)
import jax
import jax.numpy as jnp
from jax.experimental import pallas as pl


def kernel(feature, edge_index_0, edge_index_1, edge_index_2, vals_0, vals_1, vals_2, params):
    raise NotImplementedError("write your pallas kernel here")


# speedup vs baseline: 3.7998x; 3.7998x over previous
import functools

import jax
import jax.numpy as jnp
import numpy as np
from jax import lax
from jax.experimental import pallas as pl
from jax.experimental.pallas import tpu as pltpu
from jax.experimental.pallas import tpu_sc as plsc

N = 10000
E = 320000
NFEAT = 128
NHID = 64
OUT = 64
P = 3

NC = 2
NS = 16
LANES = 16
RPT = 624


@functools.lru_cache(maxsize=None)
def _make_spmm(W):
    nvec = W // LANES
    chunk = 32 if W > 128 else 128
    nchunks = E // chunk
    mesh = plsc.VectorSubcoreMesh(core_axis_name="c", subcore_axis_name="s",
                                  num_cores=NC, num_subcores=NS)
    osd = jax.ShapeDtypeStruct((N, W), jnp.float32)

    @functools.partial(
        pl.kernel,
        out_type=(osd, osd, osd, osd),
        mesh=mesh,
        compiler_params=pltpu.CompilerParams(use_tc_tiling_on_sc=False),
        scratch_types=(
            pltpu.VMEM_SHARED((N, W), jnp.float32),
            pltpu.VMEM((chunk,), jnp.int32),
            pltpu.VMEM((chunk,), jnp.int32),
            pltpu.VMEM((chunk,), jnp.float32),
            pltpu.VMEM((chunk, W), jnp.float32),
            pltpu.SemaphoreType.DMA,
        ),
    )
    def spmm_k(dst0, src0, val0, dst1, src1, val1, dst2, src2, val2,
               x0, x1, x2, out0, out1, out2a, out2b,
               acc, srcv, dstv, valv, rows, sem):
        cid = lax.axis_index("c")
        sid = lax.axis_index("s")

        def fill_zero_rows():
            zero = jnp.zeros((LANES,), jnp.float32)

            def zb(t, _):
                i = t // nvec
                j = t % nvec
                rows[i, pl.ds(j * LANES, LANES)] = zero
                return 0

            lax.fori_loop(0, 8 * nvec, zb, 0)

        def zero_acc():
            r0 = pl.multiple_of(sid * RPT, 8)
            nrep = RPT // 8 + jnp.where(sid == NS - 1, 2, 0)

            def zr(i, _):
                pltpu.sync_copy(rows.at[pl.ds(0, 8)],
                                acc.at[pl.ds(pl.multiple_of(r0 + i * 8, 8), 8)])
                return 0

            lax.fori_loop(0, nrep, zr, 0)

        def copy_out(out_ref):
            r0 = pl.multiple_of(sid * RPT, 8)

            @pl.when(sid < NS - 1)
            def _():
                pltpu.sync_copy(acc.at[pl.ds(r0, RPT)],
                                out_ref.at[pl.ds(r0, RPT)])

            @pl.when(sid == NS - 1)
            def _():
                last = (NS - 1) * RPT
                pltpu.sync_copy(acc.at[pl.ds(last, N - last)],
                                out_ref.at[pl.ds(last, N - last)])

        def run_edges(dst_e, src_e, val_e, x_e, ch_lo, ch_hi):
            nch = ch_hi - ch_lo
            nk = nch // NS + jnp.where(sid < (nch % NS), 1, 0)

            def body(k, _):
                off = pl.multiple_of((ch_lo + sid + NS * k) * chunk, chunk)
                pltpu.sync_copy(src_e.at[pl.ds(off, chunk)], srcv)
                pltpu.sync_copy(dst_e.at[pl.ds(off, chunk)], dstv)
                pltpu.sync_copy(val_e.at[pl.ds(off, chunk)], valv)
                pltpu.async_copy(x_e.at[srcv], rows, sem).wait()

                def scale(cg, _):
                    v16 = valv[pl.ds(cg * LANES, LANES)]
                    for l in range(LANES):
                        vb = jnp.broadcast_to(v16[l], (LANES,))
                        c = cg * LANES + l
                        for j in range(nvec):
                            rows[c, pl.ds(j * LANES, LANES)] = (
                                rows[c, pl.ds(j * LANES, LANES)] * vb)
                    return 0

                lax.fori_loop(0, chunk // LANES, scale, 0)
                pltpu.sync_copy(rows, acc.at[dstv], add=True)
                return 0

            lax.fori_loop(0, nk, body, 0)

        fill_zero_rows()
        zero_acc()
        plsc.subcore_barrier()

        @pl.when(cid == 0)
        def _():
            run_edges(dst0, src0, val0, x0, 0, nchunks)

        @pl.when(cid == 1)
        def _():
            run_edges(dst1, src1, val1, x1, 0, nchunks)

        plsc.subcore_barrier()

        @pl.when(cid == 0)
        def _():
            copy_out(out0)

        @pl.when(cid == 1)
        def _():
            copy_out(out1)

        fill_zero_rows()
        zero_acc()
        plsc.subcore_barrier()

        @pl.when(cid == 0)
        def _():
            run_edges(dst2, src2, val2, x2, 0, nchunks // 2)

        @pl.when(cid == 1)
        def _():
            run_edges(dst2, src2, val2, x2, nchunks // 2, nchunks)

        plsc.subcore_barrier()

        @pl.when(cid == 0)
        def _():
            copy_out(out2a)

        @pl.when(cid == 1)
        def _():
            copy_out(out2b)

    return spmm_k


BLK = 1000
GRID = N // BLK


def _telu(x):
    return x * jnp.tanh(jnp.exp(x))


def _row_spec(w):
    return pl.BlockSpec((BLK, w), lambda i: (i, 0))


def _full_spec(a, b):
    return pl.BlockSpec((a, b), lambda i: (0, 0))


def _t1_body(x, w0, w1, w2, o0, o1, o2):
    xv = x[...]
    o0[...] = jnp.dot(xv, w0[...], preferred_element_type=jnp.float32)
    o1[...] = jnp.dot(xv, w1[...], preferred_element_type=jnp.float32)
    o2[...] = jnp.dot(xv, w2[...], preferred_element_type=jnp.float32)


_t1 = pl.pallas_call(
    _t1_body,
    grid=(GRID,),
    in_specs=[_row_spec(NFEAT)] + [_full_spec(NFEAT, 192)] * 3,
    out_specs=[_row_spec(192)] * 3,
    out_shape=[jax.ShapeDtypeStruct((N, 192), jnp.float32)] * 3,
)


def _t2_body(s0, s1, s2a, s2b, wsp, wsh, bsp, bsh, o0, o1, o2, oc2):
    svals = [s0[...], s1[...], s2a[...] + s2b[...]]
    wsh_v = wsh[...]
    bsh_v = bsh[...]
    for p in range(P):
        s = svals[p]
        a = _telu(s[:, :NHID] + bsp[p:p + 1, :])
        b = _telu(s[:, NHID:2 * NHID] + bsh_v)
        op = [o0, o1, o2][p]
        op[:, :NHID] = jnp.dot(a, wsp[p * NHID:(p + 1) * NHID, :],
                               preferred_element_type=jnp.float32)
        op[:, NHID:] = jnp.dot(b, wsh_v, preferred_element_type=jnp.float32)
    oc2[...] = svals[2][:, 2 * NHID:]


_t2 = pl.pallas_call(
    _t2_body,
    grid=(GRID,),
    in_specs=[_row_spec(192)] * 4 + [
        _full_spec(P * NHID, NHID),
        _full_spec(NHID, NHID),
        _full_spec(P, NHID),
        _full_spec(1, NHID),
    ],
    out_specs=[_row_spec(2 * NHID)] * 3 + [_row_spec(NHID)],
    out_shape=[jax.ShapeDtypeStruct((N, 2 * NHID), jnp.float32)] * 3
    + [jax.ShapeDtypeStruct((N, NHID), jnp.float32)],
)


def _t3a_body(s0, s1, s2a, s2b, bsp2, bsh2,
              sp0, sp1, sp2, sh0, sh1, sh2, hsh,
              ps0, ps1, ps2, pm0, pm1, pm2, pz0, pz1, pz2):
    svals = [s0[...], s1[...], s2a[...] + s2b[...]]
    bsh = bsh2[...]
    shs = []
    for p in range(P):
        s = svals[p]
        spec = s[:, :OUT] + bsp2[p:p + 1, :]
        sh = s[:, OUT:] + bsh
        shs.append(sh)
        [sp0, sp1, sp2][p][...] = spec
        [sh0, sh1, sh2][p][...] = sh
        mxb = jnp.max(spec, axis=0)
        [ps0, ps1, ps2][p][...] = jnp.sum(spec, axis=0)[None, None, :]
        [pm0, pm1, pm2][p][...] = mxb[None, None, :]
        [pz0, pz1, pz2][p][...] = jnp.sum(
            jnp.exp(spec - mxb[None, :]), axis=0)[None, None, :]
    hsh[...] = (shs[0] + shs[1] + shs[2]) / 3.0


_t3a = pl.pallas_call(
    _t3a_body,
    grid=(GRID,),
    in_specs=[_row_spec(2 * OUT)] * 4 + [
        _full_spec(P, OUT),
        _full_spec(1, OUT),
    ],
    out_specs=[_row_spec(OUT)] * 7
    + [pl.BlockSpec((1, 1, OUT), lambda i: (i, 0, 0))] * 9,
    out_shape=[jax.ShapeDtypeStruct((N, OUT), jnp.float32)] * 7
    + [jax.ShapeDtypeStruct((GRID, 1, OUT), jnp.float32)] * 9,
)


def _t3b_body(sp0, sp1, sp2, pm0, pm1, pm2, pz0, pz1, pz2,
              c1w, c1b, c2w, c2b, hcol, pe0, pe1, pe2):
    specs = [sp0[...], sp1[...], sp2[...]]
    for p in range(P):
        pm = [pm0, pm1, pm2][p][...]
        mx1 = jnp.max(pm, axis=0)
        z1 = jnp.sum([pz0, pz1, pz2][p][...] * jnp.exp(pm - mx1[None]), axis=0)
        e = jnp.exp(specs[p] - mx1)
        prob = e / z1
        [pe0, pe1, pe2][p][...] = -jnp.sum(
            prob * jnp.log(prob + 1e-06), axis=0)[None, None, :]
    cat = jnp.concatenate(specs, axis=1)
    h = jnp.maximum(
        jnp.dot(cat, c1w[...], preferred_element_type=jnp.float32) + c1b[...], 0.0)
    hcol[...] = jnp.dot(h, c2w[...], preferred_element_type=jnp.float32) + c2b[...]


_t3b = pl.pallas_call(
    _t3b_body,
    grid=(GRID,),
    in_specs=[_row_spec(OUT)] * 3
    + [pl.BlockSpec((GRID, 1, OUT), lambda i: (0, 0, 0))] * 6 + [
        _full_spec(P * OUT, NHID),
        _full_spec(1, NHID),
        _full_spec(NHID, OUT),
        _full_spec(1, OUT),
    ],
    out_specs=[_row_spec(OUT)]
    + [pl.BlockSpec((1, 1, OUT), lambda i: (i, 0, 0))] * 3,
    out_shape=[jax.ShapeDtypeStruct((N, OUT), jnp.float32)]
    + [jax.ShapeDtypeStruct((GRID, 1, OUT), jnp.float32)] * 3,
)


def _t4_body(c0, c1, c2, wtb, b1, w2, u1, x3):
    u = (c0[...] * wtb[0:1, :] + c1[...] * wtb[1:2, :]
         + c2[...] * wtb[2:3, :] + b1[...])
    u1[...] = u
    x3[...] = jnp.dot(u, w2[...], preferred_element_type=jnp.float32)


_t4 = pl.pallas_call(
    _t4_body,
    grid=(GRID,),
    in_specs=[_row_spec(OUT)] * 3 + [
        _full_spec(P, OUT),
        _full_spec(1, OUT),
        _full_spec(OUT, OUT),
    ],
    out_specs=[_row_spec(OUT)] * 2,
    out_shape=[jax.ShapeDtypeStruct((N, OUT), jnp.float32)] * 2,
)


def _t5_body(sp0, sp1, sp2, hshr, hcolr, u1r, d0, d1, d2a, d2b,
             wtb, wpb, b2, pw, pb, outr, hrawr):
    u2 = (d0[...] * wtb[0:1, :] + d1[...] * wtb[1:2, :]
          + (d2a[...] + d2b[...]) * wtb[2:3, :] + b2[...])
    hraw = (u1r[...] + u2) * 0.5
    hrawr[...] = hraw
    hsp = (sp0[...] * wpb[0:1, :] + sp1[...] * wpb[1:2, :]
           + sp2[...] * wpb[2:3, :])
    cat = jnp.concatenate([hsp, hshr[...], hcolr[...], hraw], axis=1)
    outr[...] = jnp.dot(cat, pw[...], preferred_element_type=jnp.float32) + pb[...]


_t5 = pl.pallas_call(
    _t5_body,
    grid=(GRID,),
    in_specs=[_row_spec(OUT)] * 10 + [
        _full_spec(P, OUT),
        _full_spec(P, OUT),
        _full_spec(1, OUT),
        _full_spec(4 * OUT, OUT),
        _full_spec(1, OUT),
    ],
    out_specs=[_row_spec(OUT)] * 2,
    out_shape=[jax.ShapeDtypeStruct((N, OUT), jnp.float32)] * 2,
)


def kernel(feature, edge_index_0, edge_index_1, edge_index_2,
           vals_0, vals_1, vals_2, params):
    eis = (edge_index_0, edge_index_1, edge_index_2)
    vals = (vals_0, vals_1, vals_2)
    dsts = [ei[0] for ei in eis]
    srcs = [ei[1] for ei in eis]

    w1 = [jnp.concatenate([params["spec1_W_" + str(p)], params["sh1_W"],
                           params["raw1_W"]], axis=1) for p in range(P)]
    x1_0, x1_1, x1_2 = _t1(feature, w1[0], w1[1], w1[2])

    s1_0, s1_1, s1_2a, s1_2b = _make_spmm(192)(
        dsts[0], srcs[0], vals[0], dsts[1], srcs[1], vals[1],
        dsts[2], srcs[2], vals[2], x1_0, x1_1, x1_2)

    wsp2 = jnp.concatenate([params["spec2_W_" + str(p)] for p in range(P)], axis=0)
    bsp1 = jnp.stack([params["spec1_b_" + str(p)] for p in range(P)], axis=0)
    x2_0, x2_1, x2_2, c2sum = _t2(
        s1_0, s1_1, s1_2a, s1_2b, wsp2, params["sh2_W"], bsp1,
        params["sh1_b"][None, :])

    s2_0, s2_1, s2_2a, s2_2b = _make_spmm(128)(
        dsts[0], srcs[0], vals[0], dsts[1], srcs[1], vals[1],
        dsts[2], srcs[2], vals[2], x2_0, x2_1, x2_2)

    bsp2 = jnp.stack([params["spec2_b_" + str(p)] for p in range(P)], axis=0)
    (sp0, sp1, sp2, sh0, sh1, sh2, hsh,
     ps0, ps1, ps2, pm0, pm1, pm2, pz0, pz1, pz2) = _t3a(
        s2_0, s2_1, s2_2a, s2_2b, bsp2, params["sh2_b"][None, :])
    hcol, pe0, pe1, pe2 = _t3b(
        sp0, sp1, sp2, pm0, pm1, pm2, pz0, pz1, pz2,
        params["col1_W"], params["col1_b"][None, :],
        params["col2_W"], params["col2_b"][None, :])
    ps = jnp.stack([
        jnp.concatenate([
            jnp.sum([ps0, ps1, ps2][p], axis=(0, 1)) / N,
            jnp.max([pm0, pm1, pm2][p], axis=(0, 1)),
            jnp.sum([pe0, pe1, pe2][p], axis=(0, 1)),
        ]) for p in range(P)], axis=0)

    sim = ps @ ps.T / (np.sqrt(3.0 * OUT).astype(np.float32) * params["tau"])
    t_mat = jax.nn.softmax(sim, axis=1)
    pi0 = jax.nn.softmax(params["weight_b"].squeeze())
    pi = pi0
    for _ in range(13):
        pi = 0.2 * pi0 + 0.8 * (pi @ t_mat)
    wtb = jnp.broadcast_to(pi[:, None], (P, OUT))
    wpb = jnp.broadcast_to(jax.nn.softmax(pi)[:, None], (P, OUT))

    u1, x3 = _t4(s1_0[:, 2 * NHID:], s1_1[:, 2 * NHID:], c2sum, wtb,
                 params["raw1_b"][None, :], params["raw2_W"])

    d0, d1, d2a, d2b = _make_spmm(64)(
        dsts[0], srcs[0], vals[0], dsts[1], srcs[1], vals[1],
        dsts[2], srcs[2], vals[2], x3, x3, x3)

    out, hraw = _t5(sp0, sp1, sp2, hsh, hcol, u1, d0, d1, d2a, d2b,
                    wtb, wpb, params["raw2_b"][None, :],
                    params["proj_W"], params["proj_b"][None, :])

    return (out, sp0, sp1, sp2, sh0, sh1, sh2, hcol, hraw)


# speedup vs baseline: 7.0855x; 1.8647x over previous
import functools

import jax
import jax.numpy as jnp
import numpy as np
from jax import lax
from jax.experimental import pallas as pl
from jax.experimental.pallas import tpu as pltpu
from jax.experimental.pallas import tpu_sc as plsc

N = 10000
E = 320000
NFEAT = 128
NHID = 64
OUT = 64
P = 3

NC = 2
NS = 16
LANES = 16
CH = 128
NCH = E // CH
R = 4
RPT = 624


@functools.lru_cache(maxsize=None)
def _make_spmm(W2):
    nvec = W2 // LANES
    mesh = plsc.VectorSubcoreMesh(core_axis_name="c", subcore_axis_name="s",
                                  num_cores=NC, num_subcores=NS)
    osd = jax.ShapeDtypeStruct((N, W2), jnp.float32)

    @functools.partial(
        pl.kernel,
        out_type=(osd,) * 6,
        mesh=mesh,
        compiler_params=pltpu.CompilerParams(use_tc_tiling_on_sc=False),
        scratch_types=(
            pltpu.VMEM_SHARED((N, W2), jnp.float32),
            pltpu.VMEM((R, 3, CH), jnp.int32),
            pltpu.VMEM((R, CH, W2), jnp.float32),
            pltpu.SemaphoreType.DMA((R,)),
            pltpu.SemaphoreType.DMA((R,)),
            pltpu.SemaphoreType.DMA((R,)),
        ),
    )
    def spmm_k(epk0, epk1, epk2, xl0, xh0, xl1, xh1, xl2, xh2,
               ol0, oh0, ol1, oh1, ol2, oh2,
               acc, idxb, rows, sem_i, sem_g, sem_s):
        cid = lax.axis_index("c")
        sid = lax.axis_index("s")

        def fill_zero_rows():
            zero = jnp.zeros((LANES,), jnp.float32)

            def zb(t, _):
                i = t // nvec
                j = t % nvec
                rows[0, i, pl.ds(j * LANES, LANES)] = zero
                return 0

            lax.fori_loop(0, CH * nvec, zb, 0)

        def zero_acc():
            r0 = pl.multiple_of(sid * RPT, 8)
            nfull = RPT // CH + jnp.where(sid == NS - 1, 1, 0)

            def zf(i, _):
                pltpu.sync_copy(
                    rows.at[0],
                    acc.at[pl.ds(pl.multiple_of(r0 + i * CH, 8), CH)])
                return 0

            lax.fori_loop(0, nfull, zf, 0)

            @pl.when(sid < NS - 1)
            def _():
                rem = RPT - (RPT // CH) * CH
                pltpu.sync_copy(
                    rows.at[0, pl.ds(0, rem)],
                    acc.at[pl.ds(pl.multiple_of(r0 + (RPT // CH) * CH, 8),
                                 rem)])

        def copy_out(out_ref):
            r0 = pl.multiple_of(sid * RPT, 8)

            @pl.when(sid < NS - 1)
            def _():
                pltpu.sync_copy(acc.at[pl.ds(r0, RPT)],
                                out_ref.at[pl.ds(r0, RPT)])

            @pl.when(sid == NS - 1)
            def _():
                last = (NS - 1) * RPT
                pltpu.sync_copy(acc.at[pl.ds(last, N - last)],
                                out_ref.at[pl.ds(last, N - last)])

        def run_edges(epk_e, x_e):
            nk = NCH // NS + jnp.where(sid < (NCH % NS), 1, 0)
            ngroups = nk // R
            ntail = nk - ngroups * R

            def scale(r):
                def sc(cg, _):
                    v16 = lax.bitcast_convert_type(
                        idxb[r, 2, pl.ds(cg * LANES, LANES)], jnp.float32)
                    for l in range(LANES):
                        vb = jnp.broadcast_to(v16[l], (LANES,))
                        c = cg * LANES + l
                        for j in range(nvec):
                            rows[r, c, pl.ds(j * LANES, LANES)] = (
                                rows[r, c, pl.ds(j * LANES, LANES)] * vb)
                    return 0

                lax.fori_loop(0, CH // LANES, sc, 0)

            def grp(kg, _):
                def fa(r, _):
                    @pl.when(kg > 0)
                    def _():
                        pltpu.make_async_copy(
                            rows.at[r], acc.at[idxb.at[r, 1]],
                            sem_s.at[r]).wait()
                    g = sid + NS * (kg * R + r)
                    pltpu.async_copy(epk_e.at[g], idxb.at[r], sem_i.at[r])
                    return 0

                lax.fori_loop(0, R, fa, 0)

                def fb(r, _):
                    pltpu.make_async_copy(epk_e.at[0], idxb.at[r],
                                          sem_i.at[r]).wait()
                    pltpu.async_copy(x_e.at[idxb.at[r, 0]], rows.at[r],
                                     sem_g.at[r])
                    return 0

                lax.fori_loop(0, R, fb, 0)

                def fc(r, _):
                    pltpu.make_async_copy(x_e.at[idxb.at[r, 0]], rows.at[r],
                                          sem_g.at[r]).wait()
                    scale(r)
                    pltpu.async_copy(rows.at[r], acc.at[idxb.at[r, 1]],
                                     sem_s.at[r], add=True)
                    return 0

                lax.fori_loop(0, R, fc, 0)
                return 0

            lax.fori_loop(0, ngroups, grp, 0)

            @pl.when(ngroups > 0)
            def _():
                def fd(r, _):
                    pltpu.make_async_copy(rows.at[r], acc.at[idxb.at[r, 1]],
                                          sem_s.at[r]).wait()
                    return 0

                lax.fori_loop(0, R, fd, 0)

            def tail(t, _):
                g = sid + NS * (ngroups * R + t)
                pltpu.sync_copy(epk_e.at[g], idxb.at[0])
                pltpu.async_copy(x_e.at[idxb.at[0, 0]], rows.at[0],
                                 sem_g.at[0]).wait()
                scale(0)
                pltpu.sync_copy(rows.at[0], acc.at[idxb.at[0, 1]], add=True)
                return 0

            lax.fori_loop(0, ntail, tail, 0)

        for p, (epk_e, xl, xh, ol, oh) in enumerate((
                (epk0, xl0, xh0, ol0, oh0),
                (epk1, xl1, xh1, ol1, oh1),
                (epk2, xl2, xh2, ol2, oh2))):
            fill_zero_rows()
            zero_acc()
            plsc.subcore_barrier()

            @pl.when(cid == 0)
            def _():
                run_edges(epk_e, xl)

            @pl.when(cid == 1)
            def _():
                run_edges(epk_e, xh)

            plsc.subcore_barrier()

            @pl.when(cid == 0)
            def _():
                copy_out(ol)

            @pl.when(cid == 1)
            def _():
                copy_out(oh)

    return spmm_k


BLK = 1000
GRID = N // BLK


def _telu(x):
    return x * jnp.tanh(jnp.exp(x))


def _row_spec(w):
    return pl.BlockSpec((BLK, w), lambda i: (i, 0))


def _full_spec(a, b):
    return pl.BlockSpec((a, b), lambda i: (0, 0))


def _t1_body(x, w0, w1, w2, o0l, o0h, o1l, o1h, o2l, o2h):
    xv = x[...]
    for p, (ol, oh) in enumerate(((o0l, o0h), (o1l, o1h), (o2l, o2h))):
        o = jnp.dot(xv, [w0, w1, w2][p][...],
                    preferred_element_type=jnp.float32)
        ol[...] = o[:, :96]
        oh[...] = o[:, 96:]


_t1 = pl.pallas_call(
    _t1_body,
    grid=(GRID,),
    in_specs=[_row_spec(NFEAT)] + [_full_spec(NFEAT, 192)] * 3,
    out_specs=[_row_spec(96)] * 6,
    out_shape=[jax.ShapeDtypeStruct((N, 96), jnp.float32)] * 6,
)


def _t2_body(s0l, s0h, s1l, s1h, s2l, s2h, wsp, wsh, bsp, bsh,
             o0l, o0h, o1l, o1h, o2l, o2h):
    wsh_v = wsh[...]
    bsh_v = bsh[...]
    for p in range(P):
        s = jnp.concatenate([[s0l, s1l, s2l][p][...],
                             [s0h, s1h, s2h][p][...]], axis=1)
        a = _telu(s[:, :NHID] + bsp[p:p + 1, :])
        b = _telu(s[:, NHID:2 * NHID] + bsh_v)
        [o0l, o1l, o2l][p][...] = jnp.dot(
            a, wsp[p * NHID:(p + 1) * NHID, :],
            preferred_element_type=jnp.float32)
        [o0h, o1h, o2h][p][...] = jnp.dot(
            b, wsh_v, preferred_element_type=jnp.float32)


_t2 = pl.pallas_call(
    _t2_body,
    grid=(GRID,),
    in_specs=[_row_spec(96)] * 6 + [
        _full_spec(P * NHID, NHID),
        _full_spec(NHID, NHID),
        _full_spec(P, NHID),
        _full_spec(1, NHID),
    ],
    out_specs=[_row_spec(NHID)] * 6,
    out_shape=[jax.ShapeDtypeStruct((N, NHID), jnp.float32)] * 6,
)


def _t3a_body(s0l, s0h, s1l, s1h, s2l, s2h, bsp2, bsh2,
              sp0, sp1, sp2, sh0, sh1, sh2, hsh,
              ps0, ps1, ps2, pm0, pm1, pm2, pz0, pz1, pz2):
    bsh = bsh2[...]
    shs = []
    for p in range(P):
        spec = [s0l, s1l, s2l][p][...] + bsp2[p:p + 1, :]
        sh = [s0h, s1h, s2h][p][...] + bsh
        shs.append(sh)
        [sp0, sp1, sp2][p][...] = spec
        [sh0, sh1, sh2][p][...] = sh
        mxb = jnp.max(spec, axis=0)
        [ps0, ps1, ps2][p][...] = jnp.sum(spec, axis=0)[None, None, :]
        [pm0, pm1, pm2][p][...] = mxb[None, None, :]
        [pz0, pz1, pz2][p][...] = jnp.sum(
            jnp.exp(spec - mxb[None, :]), axis=0)[None, None, :]
    hsh[...] = (shs[0] + shs[1] + shs[2]) / 3.0


_t3a = pl.pallas_call(
    _t3a_body,
    grid=(GRID,),
    in_specs=[_row_spec(OUT)] * 6 + [
        _full_spec(P, OUT),
        _full_spec(1, OUT),
    ],
    out_specs=[_row_spec(OUT)] * 7
    + [pl.BlockSpec((1, 1, OUT), lambda i: (i, 0, 0))] * 9,
    out_shape=[jax.ShapeDtypeStruct((N, OUT), jnp.float32)] * 7
    + [jax.ShapeDtypeStruct((GRID, 1, OUT), jnp.float32)] * 9,
)


def _t3b_body(sp0, sp1, sp2, pm0, pm1, pm2, pz0, pz1, pz2,
              c1w, c1b, c2w, c2b, hcol, pe0, pe1, pe2):
    specs = [sp0[...], sp1[...], sp2[...]]
    for p in range(P):
        pm = [pm0, pm1, pm2][p][...]
        mx1 = jnp.max(pm, axis=0)
        z1 = jnp.sum([pz0, pz1, pz2][p][...] * jnp.exp(pm - mx1[None]), axis=0)
        e = jnp.exp(specs[p] - mx1)
        prob = e / z1
        [pe0, pe1, pe2][p][...] = -jnp.sum(
            prob * jnp.log(prob + 1e-06), axis=0)[None, None, :]
    cat = jnp.concatenate(specs, axis=1)
    h = jnp.maximum(
        jnp.dot(cat, c1w[...], preferred_element_type=jnp.float32) + c1b[...], 0.0)
    hcol[...] = jnp.dot(h, c2w[...], preferred_element_type=jnp.float32) + c2b[...]


_t3b = pl.pallas_call(
    _t3b_body,
    grid=(GRID,),
    in_specs=[_row_spec(OUT)] * 3
    + [pl.BlockSpec((GRID, 1, OUT), lambda i: (0, 0, 0))] * 6 + [
        _full_spec(P * OUT, NHID),
        _full_spec(1, NHID),
        _full_spec(NHID, OUT),
        _full_spec(1, OUT),
    ],
    out_specs=[_row_spec(OUT)]
    + [pl.BlockSpec((1, 1, OUT), lambda i: (i, 0, 0))] * 3,
    out_shape=[jax.ShapeDtypeStruct((N, OUT), jnp.float32)]
    + [jax.ShapeDtypeStruct((GRID, 1, OUT), jnp.float32)] * 3,
)


def _t4_body(c0, c1, c2, wtb, b1, w2, u1, x3l, x3h):
    u = (c0[...] * wtb[0:1, :] + c1[...] * wtb[1:2, :]
         + c2[...] * wtb[2:3, :] + b1[...])
    u1[...] = u
    x3 = jnp.dot(u, w2[...], preferred_element_type=jnp.float32)
    x3l[...] = x3[:, :OUT // 2]
    x3h[...] = x3[:, OUT // 2:]


_t4 = pl.pallas_call(
    _t4_body,
    grid=(GRID,),
    in_specs=[_row_spec(OUT)] * 3 + [
        _full_spec(P, OUT),
        _full_spec(1, OUT),
        _full_spec(OUT, OUT),
    ],
    out_specs=[_row_spec(OUT)] + [_row_spec(OUT // 2)] * 2,
    out_shape=[jax.ShapeDtypeStruct((N, OUT), jnp.float32)]
    + [jax.ShapeDtypeStruct((N, OUT // 2), jnp.float32)] * 2,
)


def _t5_body(sp0, sp1, sp2, hshr, hcolr, u1r, d0l, d0h, d1l, d1h, d2l, d2h,
             wtb, wpb, b2, pw, pb, outr, hrawr):
    ds = [jnp.concatenate([[d0l, d1l, d2l][p][...],
                           [d0h, d1h, d2h][p][...]], axis=1)
          for p in range(P)]
    u2 = (ds[0] * wtb[0:1, :] + ds[1] * wtb[1:2, :]
          + ds[2] * wtb[2:3, :] + b2[...])
    hraw = (u1r[...] + u2) * 0.5
    hrawr[...] = hraw
    hsp = (sp0[...] * wpb[0:1, :] + sp1[...] * wpb[1:2, :]
           + sp2[...] * wpb[2:3, :])
    cat = jnp.concatenate([hsp, hshr[...], hcolr[...], hraw], axis=1)
    outr[...] = jnp.dot(cat, pw[...], preferred_element_type=jnp.float32) + pb[...]


_t5 = pl.pallas_call(
    _t5_body,
    grid=(GRID,),
    in_specs=[_row_spec(OUT)] * 6 + [_row_spec(OUT // 2)] * 6 + [
        _full_spec(P, OUT),
        _full_spec(P, OUT),
        _full_spec(1, OUT),
        _full_spec(4 * OUT, OUT),
        _full_spec(1, OUT),
    ],
    out_specs=[_row_spec(OUT)] * 2,
    out_shape=[jax.ShapeDtypeStruct((N, OUT), jnp.float32)] * 2,
)


def _pack_edges(ei, v):
    return jnp.stack([
        ei[1].reshape(NCH, CH),
        ei[0].reshape(NCH, CH),
        lax.bitcast_convert_type(v, jnp.int32).reshape(NCH, CH),
    ], axis=1)


def kernel(feature, edge_index_0, edge_index_1, edge_index_2,
           vals_0, vals_1, vals_2, params):
    epk = [_pack_edges(ei, v) for ei, v in
           ((edge_index_0, vals_0), (edge_index_1, vals_1),
            (edge_index_2, vals_2))]

    w1 = [jnp.concatenate([params["spec1_W_" + str(p)], params["sh1_W"],
                           params["raw1_W"]], axis=1) for p in range(P)]
    x1 = _t1(feature, w1[0], w1[1], w1[2])

    s1 = _make_spmm(96)(epk[0], epk[1], epk[2], *x1)

    wsp2 = jnp.concatenate([params["spec2_W_" + str(p)] for p in range(P)],
                           axis=0)
    bsp1 = jnp.stack([params["spec1_b_" + str(p)] for p in range(P)], axis=0)
    x2 = _t2(*s1, wsp2, params["sh2_W"], bsp1, params["sh1_b"][None, :])

    s2 = _make_spmm(64)(epk[0], epk[1], epk[2], *x2)

    bsp2 = jnp.stack([params["spec2_b_" + str(p)] for p in range(P)], axis=0)
    (sp0, sp1, sp2, sh0, sh1, sh2, hsh,
     ps0, ps1, ps2, pm0, pm1, pm2, pz0, pz1, pz2) = _t3a(
        *s2, bsp2, params["sh2_b"][None, :])
    hcol, pe0, pe1, pe2 = _t3b(
        sp0, sp1, sp2, pm0, pm1, pm2, pz0, pz1, pz2,
        params["col1_W"], params["col1_b"][None, :],
        params["col2_W"], params["col2_b"][None, :])
    ps = jnp.stack([
        jnp.concatenate([
            jnp.sum([ps0, ps1, ps2][p], axis=(0, 1)) / N,
            jnp.max([pm0, pm1, pm2][p], axis=(0, 1)),
            jnp.sum([pe0, pe1, pe2][p], axis=(0, 1)),
        ]) for p in range(P)], axis=0)

    sim = ps @ ps.T / (np.sqrt(3.0 * OUT).astype(np.float32) * params["tau"])
    t_mat = jax.nn.softmax(sim, axis=1)
    pi0 = jax.nn.softmax(params["weight_b"].squeeze())
    pi = pi0
    for _ in range(13):
        pi = 0.2 * pi0 + 0.8 * (pi @ t_mat)
    wtb = jnp.broadcast_to(pi[:, None], (P, OUT))
    wpb = jnp.broadcast_to(jax.nn.softmax(pi)[:, None], (P, OUT))

    u1, x3l, x3h = _t4(s1[1][:, 32:], s1[3][:, 32:], s1[5][:, 32:], wtb,
                       params["raw1_b"][None, :], params["raw2_W"])

    d = _make_spmm(32)(epk[0], epk[1], epk[2], x3l, x3h, x3l, x3h, x3l, x3h)

    out, hraw = _t5(sp0, sp1, sp2, hsh, hcol, u1, d[0], d[1], d[2], d[3],
                    d[4], d[5], wtb, wpb, params["raw2_b"][None, :],
                    params["proj_W"], params["proj_b"][None, :])

    return (out, sp0, sp1, sp2, sh0, sh1, sh2, hcol, hraw)
